# baseline (speedup 1.0000x reference)
import functools

import jax
import jax.numpy as jnp
from jax import lax
from jax.experimental import pallas as pl
from jax.experimental.pallas import tpu as pltpu
from jax.experimental.pallas import tpu_sc as plsc

N = 10000
D = 128
E = 320000

NC = 2
NS = 16
NW = NC * NS
WIN = 128
NWIN = 80
EPW = NWIN * WIN
EPAD = NW * EPW
NPAD = 10240
DPT = NPAD // NS
RCHUNK = 128

_sc_mesh = plsc.VectorSubcoreMesh(core_axis_name="c", subcore_axis_name="s")


@functools.partial(
    pl.kernel,
    out_type=jax.ShapeDtypeStruct((NC, NPAD), jnp.float32),
    mesh=_sc_mesh,
    scratch_types=[
        pltpu.MemorySpace.VMEM_SHARED((NPAD,), jnp.float32),
        pltpu.VMEM((NWIN, 2, WIN), jnp.int32),
        pltpu.VMEM((128,), jnp.float32),
        pltpu.VMEM((DPT,), jnp.float32),
    ],
)
def _sc_degree(sd_hbm, deg_out, deg_sp, sd_v, ones_v, zeros_v):
    c = lax.axis_index("c")
    s = lax.axis_index("s")
    w = c * NS + s

    for j in range(128 // 16):
        ones_v[pl.ds(j * 16, 16)] = jnp.ones((16,), jnp.float32)

    def _zinit(i, carry):
        zeros_v[pl.ds(i * 16, 16)] = jnp.zeros((16,), jnp.float32)
        return carry

    lax.fori_loop(0, DPT // 16, _zinit, 0)

    pltpu.sync_copy(zeros_v, deg_sp.at[pl.ds(s * DPT, DPT)])
    pltpu.sync_copy(sd_hbm.at[w], sd_v)
    plsc.subcore_barrier()

    def _body(i, carry):
        pltpu.sync_copy(ones_v, deg_sp.at[sd_v.at[i, 1]], add=True)
        return carry

    lax.fori_loop(0, NWIN, _body, 0)
    plsc.subcore_barrier()
    pltpu.sync_copy(deg_sp.at[pl.ds(s * DPT, DPT)],
                    deg_out.at[c, pl.ds(s * DPT, DPT)])


@functools.partial(
    pl.kernel,
    out_type=jax.ShapeDtypeStruct((NC, NPAD, D), jnp.float32),
    mesh=_sc_mesh,
    scratch_types=[
        pltpu.MemorySpace.VMEM_SHARED((NPAD, D), jnp.float32),
        [pltpu.VMEM((2, WIN), jnp.int32) for _ in range(4)],
        [pltpu.VMEM((WIN, D), jnp.float32) for _ in range(2)],
        [pltpu.SemaphoreType.DMA for _ in range(4)],
        [pltpu.SemaphoreType.DMA for _ in range(2)],
    ],
)
def _sc_scatter(h_hbm, sd_hbm, out_hbm,
                agg_sp, sd_v, rows_v, isem, gsem):
    c = lax.axis_index("c")
    s = lax.axis_index("s")
    w = c * NS + s

    pltpu.sync_copy(sd_hbm.at[w, 0], sd_v[0])
    pltpu.async_copy(h_hbm.at[sd_v[0].at[0]], rows_v[0], gsem[0])
    pltpu.async_copy(sd_hbm.at[w, 1], sd_v[1], isem[1])

    def _zinit(i, carry):
        for j in range(D // 16):
            rows_v[1][i, pl.ds(j * 16, 16)] = jnp.zeros((16,), jnp.float32)
        return carry

    lax.fori_loop(0, RCHUNK, _zinit, 0)
    for k in range(DPT // RCHUNK):
        pltpu.sync_copy(rows_v[1], agg_sp.at[pl.ds(s * DPT + k * RCHUNK, RCHUNK)])
    plsc.subcore_barrier()

    NG4 = NWIN // 4

    def _body(g, carry):
        i = 4 * g
        for j in range(4):
            if j < 2:
                pltpu.async_copy(sd_hbm.at[w, i + j + 2], sd_v[(j + 2) % 4],
                                 isem[(j + 2) % 4])
            else:
                @pl.when(g < NG4 - 1)
                def _():
                    pltpu.async_copy(sd_hbm.at[w, i + j + 2], sd_v[(j + 2) % 4],
                                     isem[(j + 2) % 4])
            if j < 3:
                pltpu.make_async_copy(sd_hbm.at[w, i + j + 1], sd_v[j + 1],
                                      isem[j + 1]).wait()
                pltpu.async_copy(h_hbm.at[sd_v[j + 1].at[0]],
                                 rows_v[(j + 1) % 2], gsem[(j + 1) % 2])
            else:
                @pl.when(g < NG4 - 1)
                def _():
                    pltpu.make_async_copy(sd_hbm.at[w, i + 4], sd_v[0],
                                          isem[0]).wait()
                    pltpu.async_copy(h_hbm.at[sd_v[0].at[0]], rows_v[0],
                                     gsem[0])
            pltpu.make_async_copy(h_hbm.at[sd_v[j].at[0]], rows_v[j % 2],
                                  gsem[j % 2]).wait()
            pltpu.sync_copy(rows_v[j % 2], agg_sp.at[sd_v[j].at[1]], add=True)
        return carry

    lax.fori_loop(0, NG4, _body, 0)
    plsc.subcore_barrier()
    for k in range(DPT // RCHUNK):
        r0 = s * DPT + k * RCHUNK
        pltpu.sync_copy(agg_sp.at[pl.ds(r0, RCHUNK)], out_hbm.at[c, pl.ds(r0, RCHUNK)])


def _tc0_body(x_ref, w1_ref, g_ref):
    g_ref[...] = jnp.dot(x_ref[...], w1_ref[...],
                         preferred_element_type=jnp.float32)


def _tc0(x, W1):
    return pl.pallas_call(
        _tc0_body,
        out_shape=jax.ShapeDtypeStruct((N, D), jnp.float32),
    )(x, W1)


def _tc1_body(g_ref, d0_ref, d1_ref, h_ref, dis_ref):
    deg = d0_ref[...] + d1_ref[...]
    dis = jnp.where(deg > 0, lax.rsqrt(deg), 0.0)
    h_ref[...] = g_ref[...] * dis
    dis_ref[...] = dis


def _tc1(g1, d0, d1):
    return pl.pallas_call(
        _tc1_body,
        out_shape=[
            jax.ShapeDtypeStruct((N, D), jnp.float32),
            jax.ShapeDtypeStruct((N, 1), jnp.float32),
        ],
    )(g1, d0, d1)


def _tc2_body(p_ref, dis_ref, b1_ref, w2_ref, o_ref):
    dis = dis_ref[...]
    t = (p_ref[0, :N] + p_ref[1, :N]) * dis + b1_ref[...]
    o_ref[...] = jnp.dot(t, w2_ref[...], preferred_element_type=jnp.float32) * dis


def _tc2(p, dis, b1r, W2):
    return pl.pallas_call(
        _tc2_body,
        out_shape=jax.ShapeDtypeStruct((N, D), jnp.float32),
    )(p, dis, b1r, W2)


def _tc3_body(q_ref, dis_ref, b2_ref, o_ref):
    o_ref[...] = (q_ref[0, :N] + q_ref[1, :N]) * dis_ref[...] + b2_ref[...]


def _tc3(q, dis, b2r):
    return pl.pallas_call(
        _tc3_body,
        out_shape=jax.ShapeDtypeStruct((N, D), jnp.float32),
    )(q, dis, b2r)


def kernel(x, edge_index, W1, b1, W2, b2):
    npad = EPAD - E
    k = jnp.arange(npad, dtype=jnp.int32)
    pad_src = k % N
    pad_dst = N + (k % (NPAD - N))
    src = jnp.concatenate([edge_index[0], pad_src]).reshape(NW, NWIN, WIN)
    dst = jnp.concatenate([edge_index[1], pad_dst]).reshape(NW, NWIN, WIN)
    sd = jnp.stack([src, dst], axis=2)

    degp = _sc_degree(sd)
    g1 = _tc0(x, W1)
    d0 = degp[0, :N].reshape(N, 1)
    d1 = degp[1, :N].reshape(N, 1)

    h1s, dis = _tc1(g1, d0, d1)
    p = _sc_scatter(h1s, sd)
    h2s = _tc2(p, dis, b1.reshape(1, D), W2)
    q = _sc_scatter(h2s, sd)
    return _tc3(q, dis, b2.reshape(1, D))


# speedup vs baseline: 1.0012x; 1.0012x over previous
import functools

import jax
import jax.numpy as jnp
from jax import lax
from jax.experimental import pallas as pl
from jax.experimental.pallas import tpu as pltpu
from jax.experimental.pallas import tpu_sc as plsc

N = 10000
D = 128
E = 320000

NC = 2
NS = 16
NW = NC * NS
WIN = 128
NWIN = 80
EPW = NWIN * WIN
EPAD = NW * EPW
NPAD = 10240
DPT = NPAD // NS
RCHUNK = 128

_sc_mesh = plsc.VectorSubcoreMesh(core_axis_name="c", subcore_axis_name="s")


@functools.partial(
    pl.kernel,
    out_type=jax.ShapeDtypeStruct((NC, NPAD), jnp.float32),
    mesh=_sc_mesh,
    scratch_types=[
        pltpu.MemorySpace.VMEM_SHARED((NPAD,), jnp.float32),
        pltpu.VMEM((NWIN, 2, WIN), jnp.int32),
        pltpu.VMEM((128,), jnp.float32),
        pltpu.VMEM((DPT,), jnp.float32),
    ],
)
def _sc_degree(sd_hbm, deg_out, deg_sp, sd_v, ones_v, zeros_v):
    c = lax.axis_index("c")
    s = lax.axis_index("s")
    w = c * NS + s

    for j in range(128 // 16):
        ones_v[pl.ds(j * 16, 16)] = jnp.ones((16,), jnp.float32)

    def _zinit(i, carry):
        zeros_v[pl.ds(i * 16, 16)] = jnp.zeros((16,), jnp.float32)
        return carry

    lax.fori_loop(0, DPT // 16, _zinit, 0)

    pltpu.sync_copy(zeros_v, deg_sp.at[pl.ds(s * DPT, DPT)])
    pltpu.sync_copy(sd_hbm.at[w], sd_v)
    plsc.subcore_barrier()

    def _body(i, carry):
        pltpu.sync_copy(ones_v, deg_sp.at[sd_v.at[i, 1]], add=True)
        return carry

    lax.fori_loop(0, NWIN, _body, 0)
    plsc.subcore_barrier()
    pltpu.sync_copy(deg_sp.at[pl.ds(s * DPT, DPT)],
                    deg_out.at[c, pl.ds(s * DPT, DPT)])


@functools.partial(
    pl.kernel,
    out_type=jax.ShapeDtypeStruct((NC, NPAD, D), jnp.float32),
    mesh=_sc_mesh,
    scratch_types=[
        pltpu.MemorySpace.VMEM_SHARED((NPAD, D), jnp.float32),
        [pltpu.VMEM((2, WIN), jnp.int32) for _ in range(4)],
        [pltpu.VMEM((WIN, D), jnp.float32) for _ in range(2)],
        [pltpu.SemaphoreType.DMA for _ in range(4)],
        [pltpu.SemaphoreType.DMA for _ in range(2)],
    ],
)
def _sc_scatter(h_hbm, sd_hbm, out_hbm,
                agg_sp, sd_v, rows_v, isem, gsem):
    c = lax.axis_index("c")
    s = lax.axis_index("s")
    w = c * NS + s

    pltpu.sync_copy(sd_hbm.at[w, 0], sd_v[0])
    pltpu.async_copy(h_hbm.at[sd_v[0].at[0]], rows_v[0], gsem[0])
    pltpu.async_copy(sd_hbm.at[w, 1], sd_v[1], isem[1])

    def _zinit(i, carry):
        for j in range(D // 16):
            rows_v[1][i, pl.ds(j * 16, 16)] = jnp.zeros((16,), jnp.float32)
        return carry

    lax.fori_loop(0, RCHUNK, _zinit, 0)
    for k in range(DPT // RCHUNK):
        pltpu.sync_copy(rows_v[1], agg_sp.at[pl.ds(s * DPT + k * RCHUNK, RCHUNK)])
    plsc.subcore_barrier()

    NG4 = NWIN // 4

    def _body(g, carry):
        i = 4 * g
        for j in range(4):
            if j < 2:
                pltpu.async_copy(sd_hbm.at[w, i + j + 2], sd_v[(j + 2) % 4],
                                 isem[(j + 2) % 4])
            else:
                @pl.when(g < NG4 - 1)
                def _():
                    pltpu.async_copy(sd_hbm.at[w, i + j + 2], sd_v[(j + 2) % 4],
                                     isem[(j + 2) % 4])
            if j < 3:
                pltpu.make_async_copy(sd_hbm.at[w, i + j + 1], sd_v[j + 1],
                                      isem[j + 1]).wait()
                pltpu.async_copy(h_hbm.at[sd_v[j + 1].at[0]],
                                 rows_v[(j + 1) % 2], gsem[(j + 1) % 2])
            else:
                @pl.when(g < NG4 - 1)
                def _():
                    pltpu.make_async_copy(sd_hbm.at[w, i + 4], sd_v[0],
                                          isem[0]).wait()
                    pltpu.async_copy(h_hbm.at[sd_v[0].at[0]], rows_v[0],
                                     gsem[0])
            pltpu.make_async_copy(h_hbm.at[sd_v[j].at[0]], rows_v[j % 2],
                                  gsem[j % 2]).wait()
            pltpu.sync_copy(rows_v[j % 2], agg_sp.at[sd_v[j].at[1]], add=True)
        return carry

    lax.fori_loop(0, NG4, _body, 0)
    plsc.subcore_barrier()
    for k in range(DPT // RCHUNK):
        r0 = s * DPT + k * RCHUNK
        pltpu.sync_copy(agg_sp.at[pl.ds(r0, RCHUNK)], out_hbm.at[c, pl.ds(r0, RCHUNK)])


def _tc1_body(x_ref, w1_ref, d0_ref, d1_ref, h_ref, dis_ref):
    deg = d0_ref[...] + d1_ref[...]
    dis = jnp.where(deg > 0, lax.rsqrt(deg), 0.0)
    h = jnp.dot(x_ref[...], w1_ref[...], preferred_element_type=jnp.float32)
    h_ref[...] = h * dis
    dis_ref[...] = dis


def _tc1(x, W1, d0, d1):
    return pl.pallas_call(
        _tc1_body,
        out_shape=[
            jax.ShapeDtypeStruct((N, D), jnp.float32),
            jax.ShapeDtypeStruct((N, 1), jnp.float32),
        ],
    )(x, W1, d0, d1)


def _tc2_body(p_ref, dis_ref, b1_ref, w2_ref, o_ref):
    dis = dis_ref[...]
    t = (p_ref[0, :N] + p_ref[1, :N]) * dis + b1_ref[...]
    o_ref[...] = jnp.dot(t, w2_ref[...], preferred_element_type=jnp.float32) * dis


def _tc2(p, dis, b1r, W2):
    return pl.pallas_call(
        _tc2_body,
        out_shape=jax.ShapeDtypeStruct((N, D), jnp.float32),
    )(p, dis, b1r, W2)


def _tc3_body(q_ref, dis_ref, b2_ref, o_ref):
    o_ref[...] = (q_ref[0, :N] + q_ref[1, :N]) * dis_ref[...] + b2_ref[...]


def _tc3(q, dis, b2r):
    return pl.pallas_call(
        _tc3_body,
        out_shape=jax.ShapeDtypeStruct((N, D), jnp.float32),
    )(q, dis, b2r)


def kernel(x, edge_index, W1, b1, W2, b2):
    npad = EPAD - E
    k = jnp.arange(npad, dtype=jnp.int32)
    pad_src = k % N
    pad_dst = N + (k % (NPAD - N))
    src = jnp.concatenate([edge_index[0], pad_src]).reshape(NW, NWIN, WIN)
    dst = jnp.concatenate([edge_index[1], pad_dst]).reshape(NW, NWIN, WIN)
    sd = jnp.stack([src, dst], axis=2)

    degp = _sc_degree(sd)
    d0 = degp[0, :N].reshape(N, 1)
    d1 = degp[1, :N].reshape(N, 1)

    h1s, dis = _tc1(x, W1, d0, d1)
    p = _sc_scatter(h1s, sd)
    h2s = _tc2(p, dis, b1.reshape(1, D), W2)
    q = _sc_scatter(h2s, sd)
    return _tc3(q, dis, b2.reshape(1, D))


# speedup vs baseline: 1.0116x; 1.0104x over previous
import functools

import jax
import jax.numpy as jnp
from jax import lax
from jax.experimental import pallas as pl
from jax.experimental.pallas import tpu as pltpu
from jax.experimental.pallas import tpu_sc as plsc

N = 10000
D = 128
E = 320000

NC = 2
NS = 16
NW = NC * NS
WIN = 128
NWIN = 80
EPW = NWIN * WIN
EPAD = NW * EPW
NPAD = 10240
DPT = NPAD // NS
RCHUNK = 128

_sc_mesh = plsc.VectorSubcoreMesh(core_axis_name="c", subcore_axis_name="s")


@functools.partial(
    pl.kernel,
    out_type=jax.ShapeDtypeStruct((NC, NPAD), jnp.float32),
    mesh=_sc_mesh,
    scratch_types=[
        pltpu.MemorySpace.VMEM_SHARED((NPAD,), jnp.float32),
        pltpu.VMEM((NWIN, 2, WIN), jnp.int32),
        pltpu.VMEM((128,), jnp.float32),
        pltpu.VMEM((DPT,), jnp.float32),
        pltpu.SemaphoreType.DMA,
    ],
)
def _sc_degree(sd_hbm, deg_out, deg_sp, sd_v, ones_v, zeros_v, dsem):
    c = lax.axis_index("c")
    s = lax.axis_index("s")
    w = c * NS + s

    for j in range(128 // 16):
        ones_v[pl.ds(j * 16, 16)] = jnp.ones((16,), jnp.float32)

    def _zinit(i, carry):
        zeros_v[pl.ds(i * 16, 16)] = jnp.zeros((16,), jnp.float32)
        return carry

    lax.fori_loop(0, DPT // 16, _zinit, 0)

    pltpu.sync_copy(zeros_v, deg_sp.at[pl.ds(s * DPT, DPT)])
    pltpu.sync_copy(sd_hbm.at[w], sd_v)
    plsc.subcore_barrier()

    def _body(g, carry):
        i = 8 * g
        for k in range(8):
            pltpu.async_copy(ones_v, deg_sp.at[sd_v.at[i + k, 1]], dsem,
                             add=True)
        for k in range(8):
            pltpu.make_async_copy(ones_v, deg_sp.at[sd_v.at[i + k, 1]],
                                  dsem).wait()
        return carry

    lax.fori_loop(0, NWIN // 8, _body, 0)
    plsc.subcore_barrier()
    pltpu.sync_copy(deg_sp.at[pl.ds(s * DPT, DPT)],
                    deg_out.at[c, pl.ds(s * DPT, DPT)])


@functools.partial(
    pl.kernel,
    out_type=jax.ShapeDtypeStruct((NC, NPAD, D), jnp.float32),
    mesh=_sc_mesh,
    scratch_types=[
        pltpu.MemorySpace.VMEM_SHARED((NPAD, D), jnp.float32),
        [pltpu.VMEM((2, WIN), jnp.int32) for _ in range(4)],
        [pltpu.VMEM((WIN, D), jnp.float32) for _ in range(2)],
        [pltpu.SemaphoreType.DMA for _ in range(4)],
        [pltpu.SemaphoreType.DMA for _ in range(2)],
    ],
)
def _sc_scatter(h_hbm, sd_hbm, out_hbm,
                agg_sp, sd_v, rows_v, isem, gsem):
    c = lax.axis_index("c")
    s = lax.axis_index("s")
    w = c * NS + s

    pltpu.sync_copy(sd_hbm.at[w, 0], sd_v[0])
    pltpu.async_copy(h_hbm.at[sd_v[0].at[0]], rows_v[0], gsem[0])
    pltpu.async_copy(sd_hbm.at[w, 1], sd_v[1], isem[1])

    def _zinit(i, carry):
        for j in range(D // 16):
            rows_v[1][i, pl.ds(j * 16, 16)] = jnp.zeros((16,), jnp.float32)
        return carry

    lax.fori_loop(0, RCHUNK, _zinit, 0)
    for k in range(DPT // RCHUNK):
        pltpu.sync_copy(rows_v[1], agg_sp.at[pl.ds(s * DPT + k * RCHUNK, RCHUNK)])
    plsc.subcore_barrier()

    NG4 = NWIN // 4

    def _body(g, carry):
        i = 4 * g
        for j in range(4):
            if j < 2:
                pltpu.async_copy(sd_hbm.at[w, i + j + 2], sd_v[(j + 2) % 4],
                                 isem[(j + 2) % 4])
            else:
                @pl.when(g < NG4 - 1)
                def _():
                    pltpu.async_copy(sd_hbm.at[w, i + j + 2], sd_v[(j + 2) % 4],
                                     isem[(j + 2) % 4])
            if j < 3:
                pltpu.make_async_copy(sd_hbm.at[w, i + j + 1], sd_v[j + 1],
                                      isem[j + 1]).wait()
                pltpu.async_copy(h_hbm.at[sd_v[j + 1].at[0]],
                                 rows_v[(j + 1) % 2], gsem[(j + 1) % 2])
            else:
                @pl.when(g < NG4 - 1)
                def _():
                    pltpu.make_async_copy(sd_hbm.at[w, i + 4], sd_v[0],
                                          isem[0]).wait()
                    pltpu.async_copy(h_hbm.at[sd_v[0].at[0]], rows_v[0],
                                     gsem[0])
            pltpu.make_async_copy(h_hbm.at[sd_v[j].at[0]], rows_v[j % 2],
                                  gsem[j % 2]).wait()
            pltpu.sync_copy(rows_v[j % 2], agg_sp.at[sd_v[j].at[1]], add=True)
        return carry

    lax.fori_loop(0, NG4, _body, 0)
    plsc.subcore_barrier()
    for k in range(DPT // RCHUNK):
        r0 = s * DPT + k * RCHUNK
        pltpu.sync_copy(agg_sp.at[pl.ds(r0, RCHUNK)], out_hbm.at[c, pl.ds(r0, RCHUNK)])


def _tc1_body(x_ref, w1_ref, d0_ref, d1_ref, h_ref, dis_ref):
    deg = d0_ref[...] + d1_ref[...]
    dis = jnp.where(deg > 0, lax.rsqrt(deg), 0.0)
    h = jnp.dot(x_ref[...], w1_ref[...], preferred_element_type=jnp.float32)
    h_ref[...] = h * dis
    dis_ref[...] = dis


BN = 2000


def _tc1(x, W1, d0, d1):
    return pl.pallas_call(
        _tc1_body,
        grid=(N // BN,),
        in_specs=[
            pl.BlockSpec((BN, D), lambda i: (i, 0)),
            pl.BlockSpec((D, D), lambda i: (0, 0)),
            pl.BlockSpec((BN, 1), lambda i: (i, 0)),
            pl.BlockSpec((BN, 1), lambda i: (i, 0)),
        ],
        out_specs=[
            pl.BlockSpec((BN, D), lambda i: (i, 0)),
            pl.BlockSpec((BN, 1), lambda i: (i, 0)),
        ],
        out_shape=[
            jax.ShapeDtypeStruct((N, D), jnp.float32),
            jax.ShapeDtypeStruct((N, 1), jnp.float32),
        ],
    )(x, W1, d0, d1)


def _tc2_body(p_ref, dis_ref, b1_ref, w2_ref, o_ref):
    dis = dis_ref[...]
    t = (p_ref[0] + p_ref[1]) * dis + b1_ref[...]
    o_ref[...] = jnp.dot(t, w2_ref[...], preferred_element_type=jnp.float32) * dis


def _tc2(p, dis, b1r, W2):
    return pl.pallas_call(
        _tc2_body,
        grid=(N // BN,),
        in_specs=[
            pl.BlockSpec((NC, BN, D), lambda i: (0, i, 0)),
            pl.BlockSpec((BN, 1), lambda i: (i, 0)),
            pl.BlockSpec((1, D), lambda i: (0, 0)),
            pl.BlockSpec((D, D), lambda i: (0, 0)),
        ],
        out_specs=pl.BlockSpec((BN, D), lambda i: (i, 0)),
        out_shape=jax.ShapeDtypeStruct((N, D), jnp.float32),
    )(p, dis, b1r, W2)


def _tc3_body(q_ref, dis_ref, b2_ref, o_ref):
    o_ref[...] = (q_ref[0] + q_ref[1]) * dis_ref[...] + b2_ref[...]


def _tc3(q, dis, b2r):
    return pl.pallas_call(
        _tc3_body,
        grid=(N // BN,),
        in_specs=[
            pl.BlockSpec((NC, BN, D), lambda i: (0, i, 0)),
            pl.BlockSpec((BN, 1), lambda i: (i, 0)),
            pl.BlockSpec((1, D), lambda i: (0, 0)),
        ],
        out_specs=pl.BlockSpec((BN, D), lambda i: (i, 0)),
        out_shape=jax.ShapeDtypeStruct((N, D), jnp.float32),
    )(q, dis, b2r)


def kernel(x, edge_index, W1, b1, W2, b2):
    npad = EPAD - E
    k = jnp.arange(npad, dtype=jnp.int32)
    pad_src = k % N
    pad_dst = N + (k % (NPAD - N))
    src = jnp.concatenate([edge_index[0], pad_src]).reshape(NW, NWIN, WIN)
    dst = jnp.concatenate([edge_index[1], pad_dst]).reshape(NW, NWIN, WIN)
    sd = jnp.stack([src, dst], axis=2)

    degp = _sc_degree(sd)
    d0 = degp[0, :N].reshape(N, 1)
    d1 = degp[1, :N].reshape(N, 1)

    h1s, dis = _tc1(x, W1, d0, d1)
    p = _sc_scatter(h1s, sd)
    h2s = _tc2(p, dis, b1.reshape(1, D), W2)
    q = _sc_scatter(h2s, sd)
    return _tc3(q, dis, b2.reshape(1, D))
